# baseline (speedup 1.0000x reference)
import functools

import jax
import jax.numpy as jnp
from jax import lax
from jax.experimental import pallas as pl
from jax.experimental.pallas import tpu as pltpu
from jax.experimental.pallas import tpu_sc as plsc

_N = 10000
_E = 320000
_DH = 128
_NW = 32
_C = 128
_G = 4
_KC = 80
_NGRP = _KC // _G
_EPW = _KC * _C
_EPAD = _NW * _EPW
_NBUF = 2
_PF = 1
_NPAD = 10112
_RPT = _NPAD // 16
_ROWBLK = 1000
_GRID = _N // _ROWBLK


def _pre_body(x1, x2, w1, w2, be, wsn, bm, hw_ref, hn_ref):
    h = jnp.maximum(
        jnp.dot(x1[...], w1[...], preferred_element_type=jnp.float32)
        + jnp.dot(x2[...], w2[...], preferred_element_type=jnp.float32)
        + be[...], 0.0)
    hsn = jnp.dot(h, wsn[...], preferred_element_type=jnp.float32)
    hw_ref[...] = hsn[:, :_DH] + bm[...]
    hn_ref[...] = hsn[:, _DH:]


def _mid_body(hw, p0, p1, wsn, bm, hw_ref, hn_ref):
    h = jnp.maximum(hw[...] + p0[0] + p1[0], 0.0)
    hsn = jnp.dot(h, wsn[...], preferred_element_type=jnp.float32)
    hw_ref[...] = hsn[:, :_DH] + bm[...]
    hn_ref[...] = hsn[:, _DH:]


def _fin_body(hw, p0, p1, wd, bd, o_ref):
    h = jnp.maximum(hw[...] + p0[0] + p1[0], 0.0)
    o_ref[...] = jnp.dot(h, wd[...], preferred_element_type=jnp.float32) + bd[...]


_row_spec = pl.BlockSpec((_ROWBLK, _DH), lambda i: (i, 0))
_w_spec = pl.BlockSpec((_DH, _DH), lambda i: (0, 0))
_wsn_spec = pl.BlockSpec((_DH, 2 * _DH), lambda i: (0, 0))
_b_spec = pl.BlockSpec((1, _DH), lambda i: (0, 0))
_p0_spec = pl.BlockSpec((1, _ROWBLK, _DH), lambda i: (0, i, 0))
_p1_spec = pl.BlockSpec((1, _ROWBLK, _DH), lambda i: (1, i, 0))

_hh_out = (jax.ShapeDtypeStruct((_N, _DH), jnp.float32),
           jax.ShapeDtypeStruct((_N, _DH), jnp.float32))


def _tc_pre(x1, x2, w1, w2, be, wsn, bm):
    return pl.pallas_call(
        _pre_body,
        grid=(_GRID,),
        in_specs=[_row_spec, _row_spec, _w_spec, _w_spec, _b_spec, _wsn_spec, _b_spec],
        out_specs=(_row_spec, _row_spec),
        out_shape=_hh_out,
    )(x1, x2, w1, w2, be, wsn, bm)


def _tc_mid(hw, parts, wsn, bm):
    return pl.pallas_call(
        _mid_body,
        grid=(_GRID,),
        in_specs=[_row_spec, _p0_spec, _p1_spec, _wsn_spec, _b_spec],
        out_specs=(_row_spec, _row_spec),
        out_shape=_hh_out,
    )(hw, parts, parts, wsn, bm)


def _tc_fin(hw, parts, wd, bd):
    return pl.pallas_call(
        _fin_body,
        grid=(_GRID,),
        in_specs=[_row_spec, _p0_spec, _p1_spec,
                  pl.BlockSpec((_DH, 8), lambda i: (0, 0)),
                  pl.BlockSpec((1, 8), lambda i: (0, 0))],
        out_specs=pl.BlockSpec((_ROWBLK, 8), lambda i: (i, 0)),
        out_shape=jax.ShapeDtypeStruct((_N, 8), jnp.float32),
    )(hw, parts, parts, wd, bd)


def _sc_body(hn_hbm, gidx_hbm, z_hbm, out_hbm, agg, i0, i1,
             r0, r1, is0, is1, g0, g1, s0, s1):
    gidx = [i0, i1]
    isem = [is0, is1]
    rows = [r0, r1]
    gs = [g0, g1]
    ss = [s0, s1]
    cid = lax.axis_index("c")
    sid = lax.axis_index("s")
    wid = sid * 2 + cid
    base = sid * _RPT

    def stage_group(q, s):
        pltpu.async_copy(gidx_hbm.at[wid, q], gidx[s], isem[s])

    def stage_wait(s):
        pltpu.make_async_copy(gidx_hbm.at[wid, 0], gidx[s], isem[s]).wait()

    def gather_start(s, c, b):
        pltpu.async_copy(hn_hbm.at[gidx[s].at[c]], rows[b], gs[b])

    def gather_wait(b):
        pltpu.make_async_copy(hn_hbm.at[gidx[0].at[0]], rows[b], gs[b]).wait()

    def scatter_start(s, c, b):
        pltpu.async_copy(rows[b], agg.at[gidx[s].at[_G + c]], ss[b], add=True)

    def scatter_wait(b):
        pltpu.make_async_copy(rows[b], agg.at[gidx[0].at[_G]], ss[b]).wait()

    stage_group(0, 0)
    stage_wait(0)
    for b in range(_PF):
        gather_start(0, b, b)
    pltpu.sync_copy(z_hbm.at[pl.ds(base, _RPT)], agg.at[pl.ds(base, _RPT)])
    plsc.subcore_barrier()

    @pl.loop(0, _NGRP // 2)
    def _(t):
        for p in range(2):
            for c in range(_G):
                j = t * 2 * _G + p * _G + c
                b = c % _NBUF

                @pl.when(j >= 1)
                def _():
                    scatter_wait((b + 1) % _NBUF)

                if c == 1:
                    if p == 0:
                        stage_group(2 * t + 1, 1)
                    else:
                        @pl.when(t <= _NGRP // 2 - 2)
                        def _():
                            stage_group(2 * t + 2, 0)
                if c == _G - 1:
                    if p == 0:
                        stage_wait(1)
                    else:
                        @pl.when(t <= _NGRP // 2 - 2)
                        def _():
                            stage_wait(0)
                pc, cc = (p, c + _PF) if c + _PF < _G else (p + 1, c + _PF - _G)
                if pc <= 1:
                    gather_start(pc % 2, cc, (b + _PF) % _NBUF)
                else:

                    @pl.when(t <= _NGRP // 2 - 2)
                    def _():
                        gather_start(0, cc, (b + _PF) % _NBUF)

                gather_wait(b)
                scatter_start(p, c, b)

    scatter_wait((_KC - 1) % _NBUF)
    plsc.subcore_barrier()
    pltpu.sync_copy(agg.at[pl.ds(base, _RPT)],
                    out_hbm.at[cid, pl.ds(base, _RPT)])


def _sc_segsum(hn, gidx, zeros):
    kern = pl.kernel(
        _sc_body,
        out_type=jax.ShapeDtypeStruct((2, _NPAD, _DH), jnp.float32),
        mesh=plsc.VectorSubcoreMesh(core_axis_name="c", subcore_axis_name="s"),
        scratch_types=[
            pltpu.VMEM_SHARED((_NPAD, _DH), jnp.float32),
            pltpu.VMEM((2 * _G, _C), jnp.int32),
            pltpu.VMEM((2 * _G, _C), jnp.int32),
        ] + [pltpu.VMEM((_C, _DH), jnp.float32)] * _NBUF
          + [pltpu.SemaphoreType.DMA] * (2 + 2 * _NBUF),
    )
    return kern(hn, gidx, zeros)


def kernel(x, x_mask, edge_index, edge_attr, batch, W_enc, b_enc, W_self, W_nei,
           b_mp, W_dec, b_dec):
    del edge_attr, batch
    x1 = x[:, :_DH]
    x2 = x_mask[:, :_DH]
    w1 = W_enc[:_DH]
    w2 = W_enc[_DH:]
    wsn = jnp.concatenate([W_self, W_nei], axis=1)
    be = b_enc.reshape(1, _DH)
    bm = b_mp.reshape(1, _DH)
    wd = jnp.pad(W_dec, ((0, 0), (0, 8 - W_dec.shape[1])))
    bd = jnp.pad(b_dec, (0, 8 - b_dec.shape[0])).reshape(1, 8)

    pad = _EPAD - _E
    src4 = jnp.concatenate([edge_index[0], jnp.zeros((pad,), jnp.int32)]
                           ).reshape(_NW, _NGRP, _G, _C)
    dst4 = jnp.concatenate([edge_index[1], jnp.full((pad,), _N, jnp.int32)]
                           ).reshape(_NW, _NGRP, _G, _C)
    gidx = jnp.concatenate([src4, dst4], axis=2)
    zeros = jnp.zeros((_NPAD, _DH), jnp.float32)

    hw, hn = _tc_pre(x1, x2, w1, w2, be, wsn, bm)
    for _ in range(3):
        parts = _sc_segsum(hn, gidx, zeros)
        hw, hn = _tc_mid(hw, parts, wsn, bm)
    parts = _sc_segsum(hn, gidx, zeros)
    out = _tc_fin(hw, parts, wd, bd)
    return out[:, :3]


# speedup vs baseline: 3.7355x; 3.7355x over previous
import functools

import jax
import jax.numpy as jnp
from jax import lax
from jax.experimental import pallas as pl
from jax.experimental.pallas import tpu as pltpu
from jax.experimental.pallas import tpu_sc as plsc

_N = 10000
_E = 320000
_DH = 128
_NW = 32
_C = 128
_G = 4
_KC = 80
_NGRP = _KC // _G
_EPW = _KC * _C
_EPAD = _NW * _EPW
_NBUF = 2
_PF = 1
_NPAD = 10112
_RPT = _NPAD // 16
_ROWBLK = 1000
_GRID = _N // _ROWBLK


def _pre_body(x1, x2, w1, w2, be, wsn, bm, hw_ref, hn_ref):
    h = jnp.maximum(
        jnp.dot(x1[...], w1[...], preferred_element_type=jnp.float32)
        + jnp.dot(x2[...], w2[...], preferred_element_type=jnp.float32)
        + be[...], 0.0)
    hsn = jnp.dot(h, wsn[...], preferred_element_type=jnp.float32)
    hw_ref[...] = hsn[:, :_DH] + bm[...]
    hn_ref[...] = hsn[:, _DH:]


def _mid_body(hw, p0, p1, wsn, bm, hw_ref, hn_ref):
    h = jnp.maximum(hw[...] + p0[0] + p1[0], 0.0)
    hsn = jnp.dot(h, wsn[...], preferred_element_type=jnp.float32)
    hw_ref[...] = hsn[:, :_DH] + bm[...]
    hn_ref[...] = hsn[:, _DH:]


def _fin_body(hw, p0, p1, wd, bd, o_ref):
    h = jnp.maximum(hw[...] + p0[0] + p1[0], 0.0)
    o_ref[...] = jnp.dot(h, wd[...], preferred_element_type=jnp.float32) + bd[...]


_row_spec = pl.BlockSpec((_ROWBLK, _DH), lambda i: (i, 0))
_w_spec = pl.BlockSpec((_DH, _DH), lambda i: (0, 0))
_wsn_spec = pl.BlockSpec((_DH, 2 * _DH), lambda i: (0, 0))
_b_spec = pl.BlockSpec((1, _DH), lambda i: (0, 0))
_p0_spec = pl.BlockSpec((1, _ROWBLK, _DH), lambda i: (0, i, 0))
_p1_spec = pl.BlockSpec((1, _ROWBLK, _DH), lambda i: (1, i, 0))

_hh_out = (jax.ShapeDtypeStruct((_N, _DH), jnp.float32),
           jax.ShapeDtypeStruct((_N, _DH), jnp.float32))


def _tc_pre(x1, x2, w1, w2, be, wsn, bm):
    return pl.pallas_call(
        _pre_body,
        grid=(_GRID,),
        in_specs=[_row_spec, _row_spec, _w_spec, _w_spec, _b_spec, _wsn_spec, _b_spec],
        out_specs=(_row_spec, _row_spec),
        out_shape=_hh_out,
    )(x1, x2, w1, w2, be, wsn, bm)


def _tc_mid(hw, parts, wsn, bm):
    return pl.pallas_call(
        _mid_body,
        grid=(_GRID,),
        in_specs=[_row_spec, _p0_spec, _p1_spec, _wsn_spec, _b_spec],
        out_specs=(_row_spec, _row_spec),
        out_shape=_hh_out,
    )(hw, parts, parts, wsn, bm)


def _tc_fin(hw, parts, wd, bd):
    return pl.pallas_call(
        _fin_body,
        grid=(_GRID,),
        in_specs=[_row_spec, _p0_spec, _p1_spec,
                  pl.BlockSpec((_DH, 8), lambda i: (0, 0)),
                  pl.BlockSpec((1, 8), lambda i: (0, 0))],
        out_specs=pl.BlockSpec((_ROWBLK, 8), lambda i: (i, 0)),
        out_shape=jax.ShapeDtypeStruct((_N, 8), jnp.float32),
    )(hw, parts, parts, wd, bd)


def _sc_body(hn_hbm, gidx_hbm, z_hbm, out_hbm, agg, i0, i1,
             r0, r1, is0, is1, g0, g1, s0, s1):
    gidx = [i0, i1]
    isem = [is0, is1]
    rows = [r0, r1]
    gs = [g0, g1]
    ss = [s0, s1]
    cid = lax.axis_index("c")
    sid = lax.axis_index("s")
    wid = sid * 2 + cid
    base = sid * _RPT

    def stage_group(q, s):
        pltpu.async_copy(gidx_hbm.at[wid, q], gidx[s], isem[s])

    def stage_wait(s):
        pltpu.make_async_copy(gidx_hbm.at[wid, 0], gidx[s], isem[s]).wait()

    def gather_start(s, c, b):
        pltpu.async_copy(hn_hbm.at[gidx[s].at[c]], rows[b], gs[b])

    def gather_wait(b):
        pltpu.make_async_copy(hn_hbm.at[gidx[0].at[0]], rows[b], gs[b]).wait()

    def scatter_start(s, c, b):
        pltpu.async_copy(rows[b], agg.at[gidx[s].at[_G + c]], ss[b], add=True)

    def scatter_wait(b):
        pltpu.make_async_copy(rows[b], agg.at[gidx[0].at[_G]], ss[b]).wait()

    stage_group(0, 0)
    stage_wait(0)
    for b in range(_PF):
        gather_start(0, b, b)
    pltpu.sync_copy(z_hbm.at[pl.ds(base, _RPT)], agg.at[pl.ds(base, _RPT)])
    plsc.subcore_barrier()

    @pl.loop(0, _NGRP // 2)
    def _(t):
        for p in range(2):
            for c in range(_G):
                j = t * 2 * _G + p * _G + c
                b = c % _NBUF

                @pl.when(j >= 1)
                def _():
                    scatter_wait((b + 1) % _NBUF)

                if c == 1:
                    if p == 0:
                        stage_group(2 * t + 1, 1)
                    else:
                        @pl.when(t <= _NGRP // 2 - 2)
                        def _():
                            stage_group(2 * t + 2, 0)
                if c == _G - 1:
                    if p == 0:
                        stage_wait(1)
                    else:
                        @pl.when(t <= _NGRP // 2 - 2)
                        def _():
                            stage_wait(0)
                pc, cc = (p, c + _PF) if c + _PF < _G else (p + 1, c + _PF - _G)
                if pc <= 1:
                    gather_start(pc % 2, cc, (b + _PF) % _NBUF)
                else:

                    @pl.when(t <= _NGRP // 2 - 2)
                    def _():
                        gather_start(0, cc, (b + _PF) % _NBUF)

                gather_wait(b)
                scatter_start(p, c, b)

    scatter_wait((_KC - 1) % _NBUF)
    plsc.subcore_barrier()
    pltpu.sync_copy(agg.at[pl.ds(base, _RPT)],
                    out_hbm.at[cid, pl.ds(base, _RPT)])


def _sc_segsum(hn, gidx, zeros):
    kern = pl.kernel(
        _sc_body,
        out_type=jax.ShapeDtypeStruct((2, _NPAD, _DH), jnp.float32),
        mesh=plsc.VectorSubcoreMesh(core_axis_name="c", subcore_axis_name="s"),
        scratch_types=[
            pltpu.VMEM_SHARED((_NPAD, _DH), jnp.float32),
            pltpu.VMEM((2 * _G, _C), jnp.int32),
            pltpu.VMEM((2 * _G, _C), jnp.int32),
        ] + [pltpu.VMEM((_C, _DH), jnp.float32)] * _NBUF
          + [pltpu.SemaphoreType.DMA] * (2 + 2 * _NBUF),
    )
    return kern(hn, gidx, zeros)


def kernel(x, x_mask, edge_index, edge_attr, batch, W_enc, b_enc, W_self, W_nei,
           b_mp, W_dec, b_dec):
    del edge_attr, batch
    x1 = x[:, :_DH]
    x2 = x_mask[:, :_DH]
    w1 = W_enc[:_DH]
    w2 = W_enc[_DH:]
    wsn = jnp.concatenate([W_self, W_nei], axis=1)
    be = b_enc.reshape(1, _DH)
    bm = b_mp.reshape(1, _DH)
    wd = jnp.pad(W_dec, ((0, 0), (0, 8 - W_dec.shape[1])))
    bd = jnp.pad(b_dec, (0, 8 - b_dec.shape[0])).reshape(1, 8)

    pad = _EPAD - _E
    pad_src = jnp.arange(pad, dtype=jnp.int32) % _N
    pad_dst = _N + jnp.arange(pad, dtype=jnp.int32) % (_NPAD - _N)
    src4 = jnp.concatenate([edge_index[0], pad_src]).reshape(_NW, _NGRP, _G, _C)
    dst4 = jnp.concatenate([edge_index[1], pad_dst]).reshape(_NW, _NGRP, _G, _C)
    gidx = jnp.concatenate([src4, dst4], axis=2)
    zeros = jnp.zeros((_NPAD, _DH), jnp.float32)

    hw, hn = _tc_pre(x1, x2, w1, w2, be, wsn, bm)
    for _ in range(3):
        parts = _sc_segsum(hn, gidx, zeros)
        hw, hn = _tc_mid(hw, parts, wsn, bm)
    parts = _sc_segsum(hn, gidx, zeros)
    out = _tc_fin(hw, parts, wd, bd)
    return out[:, :3]
